# Initial kernel scaffold; baseline (speedup 1.0000x reference)
import jax
import jax.numpy as jnp
from jax.experimental import pallas as pl


def kernel(history, lstm_W_ih, lstm_W_hh, lstm_b_ih, lstm_b_hh, g1_Wl, g1_bl, g1_Wr, g1_br, g1_att, g1_bias, g2_Wl, g2_bl, g2_Wr, g2_br, g2_att, g2_bias, dec_W1, dec_b1, dec_W2, dec_b2, edge_index, batch, focal_agent_index):
    raise NotImplementedError("write your pallas kernel here")


# speedup vs baseline: 18.3104x; 18.3104x over previous
import functools

import jax
import jax.numpy as jnp
from jax import lax
from jax.experimental import pallas as pl
from jax.experimental.pallas import tpu as pltpu
from jax.experimental.pallas import tpu_sc as plsc

_N = 50000
_T = 20
_H = 64
_G = 1024
_HOR = 30

_NP = 51200
_JUNK = _N
_K = 128
_NCH = 416
_E2P = 16 * _K * _NCH
_RPT = _NP // 16

_GP = 1152
_PC = 64
_NPC = _NP // (32 * _PC)

_f32 = jnp.float32
_i32 = jnp.int32


def _lstm_body(hist_ref, wih_ref, whh_ref, b_ref, h_ref):
    hist = hist_ref[...]
    wih = wih_ref[...]
    whh = whh_ref[...]
    b = b_ref[...]
    bn = hist.shape[0]
    h = jnp.zeros((bn, _H), _f32)
    c = jnp.zeros((bn, _H), _f32)
    for t in range(_T):
        x0 = hist[:, 2 * t:2 * t + 1]
        x1 = hist[:, 2 * t + 1:2 * t + 2]
        gates = (x0 * wih[0:1, :] + x1 * wih[1:2, :]
                 + jnp.dot(h, whh, preferred_element_type=_f32) + b)
        i = jax.nn.sigmoid(gates[:, 0:_H])
        f = jax.nn.sigmoid(gates[:, _H:2 * _H])
        g = jnp.tanh(gates[:, 2 * _H:3 * _H])
        o = jax.nn.sigmoid(gates[:, 3 * _H:4 * _H])
        c = f * c + i * g
        h = o * jnp.tanh(c)
    h_ref[...] = h


def _lstm(hist_pad, wihT, whhT, b):
    bn = 1024
    grid = (_NP // bn,)
    return pl.pallas_call(
        _lstm_body,
        grid=grid,
        in_specs=[
            pl.BlockSpec((bn, 2 * _T), lambda i: (i, 0)),
            pl.BlockSpec((2, 4 * _H), lambda i: (0, 0)),
            pl.BlockSpec((_H, 4 * _H), lambda i: (0, 0)),
            pl.BlockSpec((1, 4 * _H), lambda i: (0, 0)),
        ],
        out_specs=pl.BlockSpec((bn, _H), lambda i: (i, 0)),
        out_shape=jax.ShapeDtypeStruct((_NP, _H), _f32),
    )(hist_pad, wihT, whhT, b)


def _xform_body(x_ref, wl_ref, wr_ref, bl_ref, br_ref, tab_ref):
    x = x_ref[...]
    xl = jnp.dot(x, wl_ref[...], preferred_element_type=_f32) + bl_ref[...]
    xr = jnp.dot(x, wr_ref[...], preferred_element_type=_f32) + br_ref[...]
    tab_ref[...] = jnp.concatenate([xl, xr], axis=1)


def _xform(x, wlT, wrT, bl, br):
    bn = 1024
    grid = (_NP // bn,)
    return pl.pallas_call(
        _xform_body,
        grid=grid,
        in_specs=[
            pl.BlockSpec((bn, _H), lambda i: (i, 0)),
            pl.BlockSpec((_H, _H), lambda i: (0, 0)),
            pl.BlockSpec((_H, _H), lambda i: (0, 0)),
            pl.BlockSpec((1, _H), lambda i: (0, 0)),
            pl.BlockSpec((1, _H), lambda i: (0, 0)),
        ],
        out_specs=pl.BlockSpec((bn, 2 * _H), lambda i: (i, 0)),
        out_shape=jax.ShapeDtypeStruct((_NP, 2 * _H), _f32),
    )(x, wlT, wrT, bl, br)


def _fin_x(nums, dens, bias):
    parts = [n / (d + 1e-16) for n, d in zip(nums, dens)]
    return jax.nn.relu(jnp.concatenate(parts, axis=1) + bias)


def _fin_xform_body(n0, n1, n2, n3, d0, d1, d2, d3, bias_ref,
                    wl_ref, wr_ref, bl_ref, br_ref, tab_ref):
    x = _fin_x([n0[...], n1[...], n2[...], n3[...]],
               [d0[...], d1[...], d2[...], d3[...]], bias_ref[...])
    xl = jnp.dot(x, wl_ref[...], preferred_element_type=_f32) + bl_ref[...]
    xr = jnp.dot(x, wr_ref[...], preferred_element_type=_f32) + br_ref[...]
    tab_ref[...] = jnp.concatenate([xl, xr], axis=1)


def _nd_specs(bn):
    return ([pl.BlockSpec((bn, 16), lambda i: (i, 0))] * 4
            + [pl.BlockSpec((bn, 1), lambda i: (i, 0))] * 4)


def _fin_xform(nums, dens, bias, wlT, wrT, bl, br):
    bn = 1024
    grid = (_NP // bn,)
    return pl.pallas_call(
        _fin_xform_body,
        grid=grid,
        in_specs=_nd_specs(bn) + [
            pl.BlockSpec((1, _H), lambda i: (0, 0)),
            pl.BlockSpec((_H, _H), lambda i: (0, 0)),
            pl.BlockSpec((_H, _H), lambda i: (0, 0)),
            pl.BlockSpec((1, _H), lambda i: (0, 0)),
            pl.BlockSpec((1, _H), lambda i: (0, 0)),
        ],
        out_specs=pl.BlockSpec((bn, 2 * _H), lambda i: (i, 0)),
        out_shape=jax.ShapeDtypeStruct((_NP, 2 * _H), _f32),
    )(*nums, *dens, bias, wlT, wrT, bl, br)


def _fin_only_body(n0, n1, n2, n3, d0, d1, d2, d3, bias_ref, x_ref):
    x = _fin_x([n0[...], n1[...], n2[...], n3[...]],
               [d0[...], d1[...], d2[...], d3[...]], bias_ref[...])
    bn = x.shape[0]
    x_ref[...] = jnp.concatenate([x, jnp.zeros((bn, _H), _f32)], axis=1)


def _fin_only(nums, dens, bias):
    bn = 1024
    grid = (_NP // bn,)
    return pl.pallas_call(
        _fin_only_body,
        grid=grid,
        in_specs=_nd_specs(bn) + [pl.BlockSpec((1, _H), lambda i: (0, 0))],
        out_specs=pl.BlockSpec((bn, 2 * _H), lambda i: (i, 0)),
        out_shape=jax.ShapeDtypeStruct((_NP, 2 * _H), _f32),
    )(*nums, *dens, bias)


def _edge_sc(tab, edges, att_flat, znum, zden, hb):
    mesh = plsc.VectorSubcoreMesh(core_axis_name="c", subcore_axis_name="s")

    @functools.partial(
        pl.kernel,
        mesh=mesh,
        compiler_params=pltpu.CompilerParams(
            needs_layout_passes=False, use_tc_tiling_on_sc=False),
        out_type=[
            jax.ShapeDtypeStruct((_NP, 16), _f32),
            jax.ShapeDtypeStruct((_NP, 16), _f32),
            jax.ShapeDtypeStruct((_NP // 16, 16), _f32),
            jax.ShapeDtypeStruct((_NP // 16, 16), _f32),
        ],
        scratch_types=[
            pltpu.VMEM((16,), _i32),
            pltpu.VMEM((16, _K), _i32),
            pltpu.VMEM((_K,), _i32),
            pltpu.VMEM((_K, 2 * _H), _f32),
            pltpu.VMEM((_K, 2 * _H), _f32),
            pltpu.VMEM((_K, 16), _f32),
            pltpu.VMEM((_K, 16), _f32),
            pltpu.VMEM((64,), _f32),
            pltpu.VMEM_SHARED((_NP, 16), _f32),
            pltpu.VMEM_SHARED((_NP // 16, 16), _f32),
            pltpu.SemaphoreType.DMA,
            pltpu.SemaphoreType.DMA,
            pltpu.SemaphoreType.DMA,
        ],
    )
    def body(tab_h, edges_h, att_h, znum_h, zden_h,
             num_a_o, num_b_o, den_a_o, den_b_o,
             rowidx, ebuf, dstdv, xsbuf, xdbuf, msgbuf, denbuf, attv,
             num_sh, den_sh, sem_a, sem_b, sem_e):
        c = lax.axis_index("c")
        s = lax.axis_index("s")
        row0 = s * _RPT
        drow0 = s * (_RPT // 16)

        pltpu.sync_copy(znum_h, num_sh.at[pl.ds(row0, _RPT)])
        pltpu.sync_copy(zden_h, den_sh.at[pl.ds(drow0, _RPT // 16)])
        pltpu.sync_copy(att_h, attv)
        plsc.subcore_barrier()

        iota = lax.iota(_i32, 16)
        colj = (hb + c) * 16
        att_a = plsc.load_gather(attv, [iota + colj])
        zero16 = jnp.zeros((16,), _f32)
        nrows = _E2P // _K

        def superchunk(g8, carry):
            base = s * _NCH + g8 * 8
            vals = (base + jnp.bitwise_and(iota, 7)
                    + jnp.where(iota >= 8, nrows, 0))
            plsc.store_scatter(rowidx, [iota], vals)
            pltpu.async_copy(edges_h.at[rowidx], ebuf, sem_e).wait()

            def sub(j, carry1):
                rdst = jnp.full((16,), 8 + j, _i32)
                cj = pltpu.async_copy(tab_h.at[ebuf.at[j]], xsbuf, sem_a)
                ci = pltpu.async_copy(tab_h.at[ebuf.at[8 + j]], xdbuf, sem_b)
                cj.wait()
                ci.wait()

                def dshift(j2, carry2):
                    ix = j2 * 16 + iota
                    v = plsc.load_gather(ebuf, [rdst, ix])
                    plsc.store_scatter(dstdv, [ix],
                                       lax.shift_right_logical(v, 4))
                    return carry2

                lax.fori_loop(0, _K // 16, dshift, 0)

                def edge(k, carry2):
                    rk = jnp.full((16,), k, _i32)
                    xj0 = plsc.load_gather(xsbuf, [rk, iota + colj])
                    xi0 = plsc.load_gather(xdbuf, [rk, iota + 64 + colj])
                    t0 = xi0 + xj0
                    l0 = jnp.sum(jnp.maximum(t0, 0.2 * t0) * att_a)
                    a0 = jnp.exp(jnp.full((16,), l0, _f32))
                    plsc.store_scatter(msgbuf, [rk, iota], xj0 * a0)
                    dk = plsc.load_gather(ebuf, [rdst, rk])
                    p0 = jnp.bitwise_and(dk, 15)
                    d = jnp.where(iota == p0, a0, zero16)
                    plsc.store_scatter(denbuf, [rk, iota], d)
                    return carry2

                lax.fori_loop(0, _K, edge, 0)
                pltpu.sync_copy(msgbuf, num_sh.at[ebuf.at[8 + j]], add=True)
                pltpu.sync_copy(denbuf, den_sh.at[dstdv], add=True)
                return carry1

            lax.fori_loop(0, 8, sub, 0)
            return carry

        lax.fori_loop(0, _NCH // 8, superchunk, 0)
        plsc.subcore_barrier()

        @pl.when(c == 0)
        def _():
            pltpu.sync_copy(num_sh.at[pl.ds(row0, _RPT)],
                            num_a_o.at[pl.ds(row0, _RPT)])
            pltpu.sync_copy(den_sh.at[pl.ds(drow0, _RPT // 16)],
                            den_a_o.at[pl.ds(drow0, _RPT // 16)])

        @pl.when(c == 1)
        def _():
            pltpu.sync_copy(num_sh.at[pl.ds(row0, _RPT)],
                            num_b_o.at[pl.ds(row0, _RPT)])
            pltpu.sync_copy(den_sh.at[pl.ds(drow0, _RPT // 16)],
                            den_b_o.at[pl.ds(drow0, _RPT // 16)])

    return body(tab, edges, att_flat, znum, zden)


def _gat_layer(tab, edges, att_flat, znum, zden):
    n0, n1, d0, d1 = _edge_sc(tab, edges, att_flat, znum, zden, 0)
    att2, _ = lax.optimization_barrier((att_flat, n0))
    n2, n3, d2, d3 = _edge_sc(tab, edges, att2, znum, zden, 2)
    nums = [n0, n1, n2, n3]
    dens = [d.reshape(_NP, 1) for d in (d0, d1, d2, d3)]
    return nums, dens


def _pool_sc(x2, batch_pad, focal, zsc):
    mesh = plsc.VectorSubcoreMesh(core_axis_name="c", subcore_axis_name="s")

    @functools.partial(
        pl.kernel,
        mesh=mesh,
        compiler_params=pltpu.CompilerParams(
            needs_layout_passes=False, use_tc_tiling_on_sc=False),
        out_type=[
            jax.ShapeDtypeStruct((_GP, 2 * _H), _f32),
            jax.ShapeDtypeStruct((_GP, 2 * _H), _f32),
            jax.ShapeDtypeStruct((_G, 2 * _H), _f32),
        ],
        scratch_types=[
            pltpu.VMEM((_PC,), _i32),
            pltpu.VMEM((_PC,), _i32),
            pltpu.VMEM((_PC, 2 * _H), _f32),
            pltpu.VMEM((32,), _i32),
            pltpu.VMEM((32, 2 * _H), _f32),
            pltpu.VMEM_SHARED((_GP, 2 * _H), _f32),
            pltpu.SemaphoreType.DMA,
        ],
    )
    def body(x2_h, batch_h, focal_h, zsc_h,
             scene0_o, scene1_o, agent_o,
             segv, rbuf, rowbuf, fidxv, agbuf, scene_sh, sem):
        c = lax.axis_index("c")
        s = lax.axis_index("s")
        wid = s * 2 + c
        rows = _GP // 16
        zrow0 = s * rows
        pltpu.sync_copy(zsc_h, scene_sh.at[pl.ds(zrow0, rows)])
        plsc.subcore_barrier()
        iota = lax.iota(_i32, 16)

        def chunk(g, carry):
            base = (wid * _NPC + g) * _PC
            pltpu.sync_copy(batch_h.at[pl.ds(base, _PC)], segv)

            def fill(j, carry2):
                ix = j * 16 + iota
                plsc.store_scatter(rbuf, [ix], base + ix)
                return carry2

            lax.fori_loop(0, _PC // 16, fill, 0)
            pltpu.async_copy(x2_h.at[rbuf], rowbuf, sem).wait()
            pltpu.sync_copy(rowbuf, scene_sh.at[segv], add=True)
            return carry

        lax.fori_loop(0, _NPC, chunk, 0)

        pltpu.sync_copy(focal_h.at[pl.ds(wid * 32, 32)], fidxv)
        pltpu.async_copy(x2_h.at[fidxv], agbuf, sem).wait()
        pltpu.sync_copy(agbuf, agent_o.at[pl.ds(wid * 32, 32)])

        plsc.subcore_barrier()

        @pl.when(c == 0)
        def _():
            pltpu.sync_copy(scene_sh.at[pl.ds(zrow0, rows)],
                            scene0_o.at[pl.ds(zrow0, rows)])

        @pl.when(c == 1)
        def _():
            pltpu.sync_copy(scene_sh.at[pl.ds(zrow0, rows)],
                            scene1_o.at[pl.ds(zrow0, rows)])

    return body(x2, batch_pad, focal, zsc)


def _dec_body(ag_ref, s0_ref, s1_ref, w1_ref, b1_ref, w2_ref, b2_ref, o_ref):
    ag = ag_ref[...]
    sc = s0_ref[...] + s1_ref[...]
    fused = jnp.concatenate([ag, sc], axis=1)
    hid = jax.nn.relu(jnp.dot(fused, w1_ref[...],
                              preferred_element_type=_f32) + b1_ref[...])
    o_ref[...] = jnp.dot(hid, w2_ref[...],
                         preferred_element_type=_f32) + b2_ref[...]


def _decoder(agent, scene0, scene1, w1T, b1, w2T, b2):
    return pl.pallas_call(
        _dec_body,
        grid=(1,),
        in_specs=[
            pl.BlockSpec((_G, _H), lambda i: (0, 0)),
            pl.BlockSpec((_G, _H), lambda i: (0, 0)),
            pl.BlockSpec((_G, _H), lambda i: (0, 0)),
            pl.BlockSpec((2 * _H, 2 * _H), lambda i: (0, 0)),
            pl.BlockSpec((1, 2 * _H), lambda i: (0, 0)),
            pl.BlockSpec((2 * _H, _HOR * 2), lambda i: (0, 0)),
            pl.BlockSpec((1, _HOR * 2), lambda i: (0, 0)),
        ],
        out_specs=pl.BlockSpec((_G, _HOR * 2), lambda i: (0, 0)),
        out_shape=jax.ShapeDtypeStruct((_G, _HOR * 2), _f32),
    )(agent, scene0, scene1, w1T, b1, w2T, b2)


def kernel(history, lstm_W_ih, lstm_W_hh, lstm_b_ih, lstm_b_hh,
           g1_Wl, g1_bl, g1_Wr, g1_br, g1_att, g1_bias,
           g2_Wl, g2_bl, g2_Wr, g2_br, g2_att, g2_bias,
           dec_W1, dec_b1, dec_W2, dec_b2,
           edge_index, batch, focal_agent_index):
    f32 = _f32

    hist = history.reshape(_N, 2 * _T).astype(f32)
    hist = jnp.pad(hist, ((0, _NP - _N), (0, 0)))
    b_lstm = (lstm_b_ih + lstm_b_hh).reshape(1, 4 * _H).astype(f32)
    h = _lstm(hist, lstm_W_ih.T.astype(f32), lstm_W_hh.T.astype(f32), b_lstm)

    npad = _E2P - _N - edge_index.shape[1]
    src = jnp.concatenate([
        edge_index[0].astype(_i32),
        jnp.arange(_N, dtype=_i32),
        jnp.full((npad,), _JUNK, _i32),
    ])
    dst = jnp.concatenate([
        edge_index[1].astype(_i32),
        jnp.arange(_N, dtype=_i32),
        jnp.full((npad,), _JUNK, _i32),
    ])
    edges = jnp.concatenate([src, dst]).reshape(2 * _E2P // _K, _K)

    znum = jnp.zeros((_RPT, 16), f32)
    zden = jnp.zeros((_RPT // 16, 16), f32)

    tab1 = _xform(
        h, g1_Wl.T.astype(f32), g1_Wr.T.astype(f32),
        g1_bl.reshape(1, _H).astype(f32), g1_br.reshape(1, _H).astype(f32))
    nums1, dens1 = _gat_layer(tab1, edges, g1_att.reshape(-1).astype(f32),
                              znum, zden)

    tab2 = _fin_xform(
        nums1, dens1, g1_bias.reshape(1, _H).astype(f32),
        g2_Wl.T.astype(f32), g2_Wr.T.astype(f32),
        g2_bl.reshape(1, _H).astype(f32), g2_br.reshape(1, _H).astype(f32))

    nums2, dens2 = _gat_layer(tab2, edges, g2_att.reshape(-1).astype(f32),
                              znum, zden)
    x2 = _fin_only(nums2, dens2, g2_bias.reshape(1, _H).astype(f32))

    batch_pad = jnp.concatenate([
        batch.astype(_i32), jnp.full((_NP - _N,), _G, _i32)])
    zsc = jnp.zeros((_GP // 16, 2 * _H), f32)
    scene0, scene1, agent = _pool_sc(
        x2, batch_pad, focal_agent_index.astype(_i32), zsc)

    coords = _decoder(
        agent[:, :_H], scene0[:_G, :_H], scene1[:_G, :_H],
        dec_W1.T.astype(f32), dec_b1.reshape(1, -1).astype(f32),
        dec_W2.T.astype(f32), dec_b2.reshape(1, -1).astype(f32))
    return coords.reshape(_G, _HOR, 2)


# speedup vs baseline: 22.9151x; 1.2515x over previous
import functools

import jax
import jax.numpy as jnp
from jax import lax
from jax.experimental import pallas as pl
from jax.experimental.pallas import tpu as pltpu
from jax.experimental.pallas import tpu_sc as plsc

_N = 50000
_T = 20
_H = 64
_G = 1024
_HOR = 30

_NP = 51200
_JUNK = _N
_K = 128
_NCH = 416
_E2P = 16 * _K * _NCH
_RPT = _NP // 16

_GP = 1152
_PC = 64
_NPC = _NP // (32 * _PC)

_f32 = jnp.float32
_i32 = jnp.int32


def _lstm_body(hist_ref, wih_ref, whh_ref, b_ref, h_ref):
    hist = hist_ref[...]
    wih = wih_ref[...]
    whh = whh_ref[...]
    b = b_ref[...]
    bn = hist.shape[0]
    h = jnp.zeros((bn, _H), _f32)
    c = jnp.zeros((bn, _H), _f32)
    for t in range(_T):
        x0 = hist[:, 2 * t:2 * t + 1]
        x1 = hist[:, 2 * t + 1:2 * t + 2]
        gates = (x0 * wih[0:1, :] + x1 * wih[1:2, :]
                 + jnp.dot(h, whh, preferred_element_type=_f32) + b)
        i = jax.nn.sigmoid(gates[:, 0:_H])
        f = jax.nn.sigmoid(gates[:, _H:2 * _H])
        g = jnp.tanh(gates[:, 2 * _H:3 * _H])
        o = jax.nn.sigmoid(gates[:, 3 * _H:4 * _H])
        c = f * c + i * g
        h = o * jnp.tanh(c)
    h_ref[...] = h


def _lstm(hist_pad, wihT, whhT, b):
    bn = 1024
    grid = (_NP // bn,)
    return pl.pallas_call(
        _lstm_body,
        grid=grid,
        in_specs=[
            pl.BlockSpec((bn, 2 * _T), lambda i: (i, 0)),
            pl.BlockSpec((2, 4 * _H), lambda i: (0, 0)),
            pl.BlockSpec((_H, 4 * _H), lambda i: (0, 0)),
            pl.BlockSpec((1, 4 * _H), lambda i: (0, 0)),
        ],
        out_specs=pl.BlockSpec((bn, _H), lambda i: (i, 0)),
        out_shape=jax.ShapeDtypeStruct((_NP, _H), _f32),
    )(hist_pad, wihT, whhT, b)


def _xform_body(x_ref, wl_ref, wr_ref, bl_ref, br_ref, tab_ref):
    x = x_ref[...]
    xl = jnp.dot(x, wl_ref[...], preferred_element_type=_f32) + bl_ref[...]
    xr = jnp.dot(x, wr_ref[...], preferred_element_type=_f32) + br_ref[...]
    tab_ref[...] = jnp.concatenate([xl, xr], axis=1)


def _xform(x, wlT, wrT, bl, br):
    bn = 1024
    grid = (_NP // bn,)
    return pl.pallas_call(
        _xform_body,
        grid=grid,
        in_specs=[
            pl.BlockSpec((bn, _H), lambda i: (i, 0)),
            pl.BlockSpec((_H, _H), lambda i: (0, 0)),
            pl.BlockSpec((_H, _H), lambda i: (0, 0)),
            pl.BlockSpec((1, _H), lambda i: (0, 0)),
            pl.BlockSpec((1, _H), lambda i: (0, 0)),
        ],
        out_specs=pl.BlockSpec((bn, 2 * _H), lambda i: (i, 0)),
        out_shape=jax.ShapeDtypeStruct((_NP, 2 * _H), _f32),
    )(x, wlT, wrT, bl, br)


def _fin_x(nums, dens, bias):
    parts = [n / (d + 1e-16) for n, d in zip(nums, dens)]
    return jax.nn.relu(jnp.concatenate(parts, axis=1) + bias)


def _fin_xform_body(n0, n1, n2, n3, d0, d1, d2, d3, bias_ref,
                    wl_ref, wr_ref, bl_ref, br_ref, tab_ref):
    x = _fin_x([n0[...], n1[...], n2[...], n3[...]],
               [d0[...], d1[...], d2[...], d3[...]], bias_ref[...])
    xl = jnp.dot(x, wl_ref[...], preferred_element_type=_f32) + bl_ref[...]
    xr = jnp.dot(x, wr_ref[...], preferred_element_type=_f32) + br_ref[...]
    tab_ref[...] = jnp.concatenate([xl, xr], axis=1)


def _nd_specs(bn):
    return ([pl.BlockSpec((bn, 16), lambda i: (i, 0))] * 4
            + [pl.BlockSpec((bn, 1), lambda i: (i, 0))] * 4)


def _fin_xform(nums, dens, bias, wlT, wrT, bl, br):
    bn = 1024
    grid = (_NP // bn,)
    return pl.pallas_call(
        _fin_xform_body,
        grid=grid,
        in_specs=_nd_specs(bn) + [
            pl.BlockSpec((1, _H), lambda i: (0, 0)),
            pl.BlockSpec((_H, _H), lambda i: (0, 0)),
            pl.BlockSpec((_H, _H), lambda i: (0, 0)),
            pl.BlockSpec((1, _H), lambda i: (0, 0)),
            pl.BlockSpec((1, _H), lambda i: (0, 0)),
        ],
        out_specs=pl.BlockSpec((bn, 2 * _H), lambda i: (i, 0)),
        out_shape=jax.ShapeDtypeStruct((_NP, 2 * _H), _f32),
    )(*nums, *dens, bias, wlT, wrT, bl, br)


def _fin_only_body(n0, n1, n2, n3, d0, d1, d2, d3, bias_ref, x_ref):
    x = _fin_x([n0[...], n1[...], n2[...], n3[...]],
               [d0[...], d1[...], d2[...], d3[...]], bias_ref[...])
    bn = x.shape[0]
    x_ref[...] = jnp.concatenate([x, jnp.zeros((bn, _H), _f32)], axis=1)


def _fin_only(nums, dens, bias):
    bn = 1024
    grid = (_NP // bn,)
    return pl.pallas_call(
        _fin_only_body,
        grid=grid,
        in_specs=_nd_specs(bn) + [pl.BlockSpec((1, _H), lambda i: (0, 0))],
        out_specs=pl.BlockSpec((bn, 2 * _H), lambda i: (i, 0)),
        out_shape=jax.ShapeDtypeStruct((_NP, 2 * _H), _f32),
    )(*nums, *dens, bias)


def _edge_sc(tab, edges, att_flat, znum, zden, hb):
    mesh = plsc.VectorSubcoreMesh(core_axis_name="c", subcore_axis_name="s")

    @functools.partial(
        pl.kernel,
        mesh=mesh,
        compiler_params=pltpu.CompilerParams(
            needs_layout_passes=False, use_tc_tiling_on_sc=False),
        out_type=[
            jax.ShapeDtypeStruct((_NP, 16), _f32),
            jax.ShapeDtypeStruct((_NP, 16), _f32),
            jax.ShapeDtypeStruct((_NP // 16, 16), _f32),
            jax.ShapeDtypeStruct((_NP // 16, 16), _f32),
        ],
        scratch_types=[
            pltpu.VMEM((16,), _i32),
            pltpu.VMEM((16, _K), _i32),
            pltpu.VMEM((_K,), _i32),
            pltpu.VMEM((_K, 2 * _H), _f32),
            pltpu.VMEM((_K, 2 * _H), _f32),
            pltpu.VMEM((_K, 2 * _H), _f32),
            pltpu.VMEM((_K, 2 * _H), _f32),
            pltpu.VMEM((_K, 16), _f32),
            pltpu.VMEM((_K, 16), _f32),
            pltpu.VMEM((64,), _f32),
            pltpu.VMEM_SHARED((_NP, 16), _f32),
            pltpu.VMEM_SHARED((_NP // 16, 16), _f32),
            pltpu.SemaphoreType.DMA,
            pltpu.SemaphoreType.DMA,
            pltpu.SemaphoreType.DMA,
            pltpu.SemaphoreType.DMA,
            pltpu.SemaphoreType.DMA,
        ],
    )
    def body(tab_h, edges_h, att_h, znum_h, zden_h,
             num_a_o, num_b_o, den_a_o, den_b_o,
             rowidx, ebuf, dstdv, xsbuf0, xdbuf0, xsbuf1, xdbuf1,
             msgbuf, denbuf, attv,
             num_sh, den_sh, sem_a0, sem_b0, sem_a1, sem_b1, sem_e):
        c = lax.axis_index("c")
        s = lax.axis_index("s")
        row0 = s * _RPT
        drow0 = s * (_RPT // 16)

        pltpu.sync_copy(znum_h, num_sh.at[pl.ds(row0, _RPT)])
        pltpu.sync_copy(zden_h, den_sh.at[pl.ds(drow0, _RPT // 16)])
        pltpu.sync_copy(att_h, attv)
        plsc.subcore_barrier()

        iota = lax.iota(_i32, 16)
        colj = (hb + c) * 16
        att_a = plsc.load_gather(attv, [iota + colj])
        zero16 = jnp.zeros((16,), _f32)
        nrows = _E2P // _K

        def superchunk(g8, carry):
            base = s * _NCH + g8 * 8
            vals = (base + jnp.bitwise_and(iota, 7)
                    + jnp.where(iota >= 8, nrows, 0))
            plsc.store_scatter(rowidx, [iota], vals)
            pltpu.async_copy(edges_h.at[rowidx], ebuf, sem_e).wait()

            bufs = [(xsbuf0, xdbuf0, sem_a0, sem_b0),
                    (xsbuf1, xdbuf1, sem_a1, sem_b1)]

            def issue(j):
                xs, xd, sa, sb = bufs[j % 2]
                cj = pltpu.async_copy(tab_h.at[ebuf.at[j]], xs, sa)
                ci = pltpu.async_copy(tab_h.at[ebuf.at[8 + j]], xd, sb)
                return cj, ci

            pending = issue(0)
            for j in range(8):
                xs, xd, _, _ = bufs[j % 2]
                rdst = jnp.full((16,), 8 + j, _i32)
                pending[0].wait()
                pending[1].wait()
                if j < 7:
                    pending = issue(j + 1)

                def dshift(j2, carry2, rdst=rdst):
                    ix = j2 * 16 + iota
                    v = plsc.load_gather(ebuf, [rdst, ix])
                    plsc.store_scatter(dstdv, [ix],
                                       lax.shift_right_logical(v, 4))
                    return carry2

                lax.fori_loop(0, _K // 16, dshift, 0)

                def edge(k, carry2, xs=xs, xd=xd, rdst=rdst):
                    rk = jnp.full((16,), k, _i32)
                    xj0 = plsc.load_gather(xs, [rk, iota + colj])
                    xi0 = plsc.load_gather(xd, [rk, iota + 64 + colj])
                    t0 = xi0 + xj0
                    l0 = jnp.sum(jnp.maximum(t0, 0.2 * t0) * att_a)
                    a0 = jnp.exp(jnp.full((16,), l0, _f32))
                    plsc.store_scatter(msgbuf, [rk, iota], xj0 * a0)
                    dk = plsc.load_gather(ebuf, [rdst, rk])
                    p0 = jnp.bitwise_and(dk, 15)
                    d = jnp.where(iota == p0, a0, zero16)
                    plsc.store_scatter(denbuf, [rk, iota], d)
                    return carry2

                lax.fori_loop(0, _K, edge, 0)
                pltpu.sync_copy(msgbuf, num_sh.at[ebuf.at[8 + j]], add=True)
                pltpu.sync_copy(denbuf, den_sh.at[dstdv], add=True)
            return carry

        lax.fori_loop(0, _NCH // 8, superchunk, 0)
        plsc.subcore_barrier()

        @pl.when(c == 0)
        def _():
            pltpu.sync_copy(num_sh.at[pl.ds(row0, _RPT)],
                            num_a_o.at[pl.ds(row0, _RPT)])
            pltpu.sync_copy(den_sh.at[pl.ds(drow0, _RPT // 16)],
                            den_a_o.at[pl.ds(drow0, _RPT // 16)])

        @pl.when(c == 1)
        def _():
            pltpu.sync_copy(num_sh.at[pl.ds(row0, _RPT)],
                            num_b_o.at[pl.ds(row0, _RPT)])
            pltpu.sync_copy(den_sh.at[pl.ds(drow0, _RPT // 16)],
                            den_b_o.at[pl.ds(drow0, _RPT // 16)])

    return body(tab, edges, att_flat, znum, zden)


def _gat_layer(tab, edges, att_flat, znum, zden):
    n0, n1, d0, d1 = _edge_sc(tab, edges, att_flat, znum, zden, 0)
    att2, _ = lax.optimization_barrier((att_flat, n0))
    n2, n3, d2, d3 = _edge_sc(tab, edges, att2, znum, zden, 2)
    nums = [n0, n1, n2, n3]
    dens = [d.reshape(_NP, 1) for d in (d0, d1, d2, d3)]
    return nums, dens


def _pool_sc(x2, batch_pad, focal, zsc):
    mesh = plsc.VectorSubcoreMesh(core_axis_name="c", subcore_axis_name="s")

    @functools.partial(
        pl.kernel,
        mesh=mesh,
        compiler_params=pltpu.CompilerParams(
            needs_layout_passes=False, use_tc_tiling_on_sc=False),
        out_type=[
            jax.ShapeDtypeStruct((_GP, 2 * _H), _f32),
            jax.ShapeDtypeStruct((_GP, 2 * _H), _f32),
            jax.ShapeDtypeStruct((_G, 2 * _H), _f32),
        ],
        scratch_types=[
            pltpu.VMEM((_PC,), _i32),
            pltpu.VMEM((_PC,), _i32),
            pltpu.VMEM((_PC, 2 * _H), _f32),
            pltpu.VMEM((32,), _i32),
            pltpu.VMEM((32, 2 * _H), _f32),
            pltpu.VMEM_SHARED((_GP, 2 * _H), _f32),
            pltpu.SemaphoreType.DMA,
        ],
    )
    def body(x2_h, batch_h, focal_h, zsc_h,
             scene0_o, scene1_o, agent_o,
             segv, rbuf, rowbuf, fidxv, agbuf, scene_sh, sem):
        c = lax.axis_index("c")
        s = lax.axis_index("s")
        wid = s * 2 + c
        rows = _GP // 16
        zrow0 = s * rows
        pltpu.sync_copy(zsc_h, scene_sh.at[pl.ds(zrow0, rows)])
        plsc.subcore_barrier()
        iota = lax.iota(_i32, 16)

        def chunk(g, carry):
            base = (wid * _NPC + g) * _PC
            pltpu.sync_copy(batch_h.at[pl.ds(base, _PC)], segv)

            def fill(j, carry2):
                ix = j * 16 + iota
                plsc.store_scatter(rbuf, [ix], base + ix)
                return carry2

            lax.fori_loop(0, _PC // 16, fill, 0)
            pltpu.async_copy(x2_h.at[rbuf], rowbuf, sem).wait()
            pltpu.sync_copy(rowbuf, scene_sh.at[segv], add=True)
            return carry

        lax.fori_loop(0, _NPC, chunk, 0)

        pltpu.sync_copy(focal_h.at[pl.ds(wid * 32, 32)], fidxv)
        pltpu.async_copy(x2_h.at[fidxv], agbuf, sem).wait()
        pltpu.sync_copy(agbuf, agent_o.at[pl.ds(wid * 32, 32)])

        plsc.subcore_barrier()

        @pl.when(c == 0)
        def _():
            pltpu.sync_copy(scene_sh.at[pl.ds(zrow0, rows)],
                            scene0_o.at[pl.ds(zrow0, rows)])

        @pl.when(c == 1)
        def _():
            pltpu.sync_copy(scene_sh.at[pl.ds(zrow0, rows)],
                            scene1_o.at[pl.ds(zrow0, rows)])

    return body(x2, batch_pad, focal, zsc)


def _dec_body(ag_ref, s0_ref, s1_ref, w1_ref, b1_ref, w2_ref, b2_ref, o_ref):
    ag = ag_ref[...]
    sc = s0_ref[...] + s1_ref[...]
    fused = jnp.concatenate([ag, sc], axis=1)
    hid = jax.nn.relu(jnp.dot(fused, w1_ref[...],
                              preferred_element_type=_f32) + b1_ref[...])
    o_ref[...] = jnp.dot(hid, w2_ref[...],
                         preferred_element_type=_f32) + b2_ref[...]


def _decoder(agent, scene0, scene1, w1T, b1, w2T, b2):
    return pl.pallas_call(
        _dec_body,
        grid=(1,),
        in_specs=[
            pl.BlockSpec((_G, _H), lambda i: (0, 0)),
            pl.BlockSpec((_G, _H), lambda i: (0, 0)),
            pl.BlockSpec((_G, _H), lambda i: (0, 0)),
            pl.BlockSpec((2 * _H, 2 * _H), lambda i: (0, 0)),
            pl.BlockSpec((1, 2 * _H), lambda i: (0, 0)),
            pl.BlockSpec((2 * _H, _HOR * 2), lambda i: (0, 0)),
            pl.BlockSpec((1, _HOR * 2), lambda i: (0, 0)),
        ],
        out_specs=pl.BlockSpec((_G, _HOR * 2), lambda i: (0, 0)),
        out_shape=jax.ShapeDtypeStruct((_G, _HOR * 2), _f32),
    )(agent, scene0, scene1, w1T, b1, w2T, b2)


def kernel(history, lstm_W_ih, lstm_W_hh, lstm_b_ih, lstm_b_hh,
           g1_Wl, g1_bl, g1_Wr, g1_br, g1_att, g1_bias,
           g2_Wl, g2_bl, g2_Wr, g2_br, g2_att, g2_bias,
           dec_W1, dec_b1, dec_W2, dec_b2,
           edge_index, batch, focal_agent_index):
    f32 = _f32

    hist = history.reshape(_N, 2 * _T).astype(f32)
    hist = jnp.pad(hist, ((0, _NP - _N), (0, 0)))
    b_lstm = (lstm_b_ih + lstm_b_hh).reshape(1, 4 * _H).astype(f32)
    h = _lstm(hist, lstm_W_ih.T.astype(f32), lstm_W_hh.T.astype(f32), b_lstm)

    npad = _E2P - _N - edge_index.shape[1]
    src = jnp.concatenate([
        edge_index[0].astype(_i32),
        jnp.arange(_N, dtype=_i32),
        jnp.full((npad,), _JUNK, _i32),
    ])
    dst = jnp.concatenate([
        edge_index[1].astype(_i32),
        jnp.arange(_N, dtype=_i32),
        jnp.full((npad,), _JUNK, _i32),
    ])
    edges = jnp.concatenate([src, dst]).reshape(2 * _E2P // _K, _K)

    znum = jnp.zeros((_RPT, 16), f32)
    zden = jnp.zeros((_RPT // 16, 16), f32)

    tab1 = _xform(
        h, g1_Wl.T.astype(f32), g1_Wr.T.astype(f32),
        g1_bl.reshape(1, _H).astype(f32), g1_br.reshape(1, _H).astype(f32))
    nums1, dens1 = _gat_layer(tab1, edges, g1_att.reshape(-1).astype(f32),
                              znum, zden)

    tab2 = _fin_xform(
        nums1, dens1, g1_bias.reshape(1, _H).astype(f32),
        g2_Wl.T.astype(f32), g2_Wr.T.astype(f32),
        g2_bl.reshape(1, _H).astype(f32), g2_br.reshape(1, _H).astype(f32))

    nums2, dens2 = _gat_layer(tab2, edges, g2_att.reshape(-1).astype(f32),
                              znum, zden)
    x2 = _fin_only(nums2, dens2, g2_bias.reshape(1, _H).astype(f32))

    batch_pad = jnp.concatenate([
        batch.astype(_i32), jnp.full((_NP - _N,), _G, _i32)])
    zsc = jnp.zeros((_GP // 16, 2 * _H), f32)
    scene0, scene1, agent = _pool_sc(
        x2, batch_pad, focal_agent_index.astype(_i32), zsc)

    coords = _decoder(
        agent[:, :_H], scene0[:_G, :_H], scene1[:_G, :_H],
        dec_W1.T.astype(f32), dec_b1.reshape(1, -1).astype(f32),
        dec_W2.T.astype(f32), dec_b2.reshape(1, -1).astype(f32))
    return coords.reshape(_G, _HOR, 2)


# speedup vs baseline: 23.8586x; 1.0412x over previous
import functools

import jax
import jax.numpy as jnp
from jax import lax
from jax.experimental import pallas as pl
from jax.experimental.pallas import tpu as pltpu
from jax.experimental.pallas import tpu_sc as plsc

_N = 50000
_T = 20
_H = 64
_G = 1024
_HOR = 30

_NP = 51200
_JUNK = _N
_K = 128
_NCH = 416
_E2P = 16 * _K * _NCH
_RPT = _NP // 16

_GP = 1152
_PC = 64
_NPC = _NP // (32 * _PC)

_f32 = jnp.float32
_i32 = jnp.int32


def _lstm_body(hist_ref, wih_ref, whh_ref, b_ref, h_ref):
    hist = hist_ref[...]
    wih = wih_ref[...]
    whh = whh_ref[...]
    b = b_ref[...]
    bn = hist.shape[0]
    h = jnp.zeros((bn, _H), _f32)
    c = jnp.zeros((bn, _H), _f32)
    for t in range(_T):
        x0 = hist[:, 2 * t:2 * t + 1]
        x1 = hist[:, 2 * t + 1:2 * t + 2]
        gates = (x0 * wih[0:1, :] + x1 * wih[1:2, :]
                 + jnp.dot(h, whh, preferred_element_type=_f32) + b)
        i = jax.nn.sigmoid(gates[:, 0:_H])
        f = jax.nn.sigmoid(gates[:, _H:2 * _H])
        g = jnp.tanh(gates[:, 2 * _H:3 * _H])
        o = jax.nn.sigmoid(gates[:, 3 * _H:4 * _H])
        c = f * c + i * g
        h = o * jnp.tanh(c)
    h_ref[...] = h


def _lstm(hist_pad, wihT, whhT, b):
    bn = 1024
    grid = (_NP // bn,)
    return pl.pallas_call(
        _lstm_body,
        grid=grid,
        in_specs=[
            pl.BlockSpec((bn, 2 * _T), lambda i: (i, 0)),
            pl.BlockSpec((2, 4 * _H), lambda i: (0, 0)),
            pl.BlockSpec((_H, 4 * _H), lambda i: (0, 0)),
            pl.BlockSpec((1, 4 * _H), lambda i: (0, 0)),
        ],
        out_specs=pl.BlockSpec((bn, _H), lambda i: (i, 0)),
        out_shape=jax.ShapeDtypeStruct((_NP, _H), _f32),
    )(hist_pad, wihT, whhT, b)


def _xform_body(x_ref, wl_ref, wr_ref, bl_ref, br_ref, tab_ref):
    x = x_ref[...]
    xl = jnp.dot(x, wl_ref[...], preferred_element_type=_f32) + bl_ref[...]
    xr = jnp.dot(x, wr_ref[...], preferred_element_type=_f32) + br_ref[...]
    tab_ref[...] = jnp.concatenate([xl, xr], axis=1)


def _xform(x, wlT, wrT, bl, br):
    bn = 1024
    grid = (_NP // bn,)
    return pl.pallas_call(
        _xform_body,
        grid=grid,
        in_specs=[
            pl.BlockSpec((bn, _H), lambda i: (i, 0)),
            pl.BlockSpec((_H, _H), lambda i: (0, 0)),
            pl.BlockSpec((_H, _H), lambda i: (0, 0)),
            pl.BlockSpec((1, _H), lambda i: (0, 0)),
            pl.BlockSpec((1, _H), lambda i: (0, 0)),
        ],
        out_specs=pl.BlockSpec((bn, 2 * _H), lambda i: (i, 0)),
        out_shape=jax.ShapeDtypeStruct((_NP, 2 * _H), _f32),
    )(x, wlT, wrT, bl, br)


def _fin_x(nums, dens, bias):
    parts = [n / (d + 1e-16) for n, d in zip(nums, dens)]
    return jax.nn.relu(jnp.concatenate(parts, axis=1) + bias)


def _fin_xform_body(n0, n1, n2, n3, d0, d1, d2, d3, bias_ref,
                    wl_ref, wr_ref, bl_ref, br_ref, tab_ref):
    x = _fin_x([n0[...], n1[...], n2[...], n3[...]],
               [d0[...], d1[...], d2[...], d3[...]], bias_ref[...])
    xl = jnp.dot(x, wl_ref[...], preferred_element_type=_f32) + bl_ref[...]
    xr = jnp.dot(x, wr_ref[...], preferred_element_type=_f32) + br_ref[...]
    tab_ref[...] = jnp.concatenate([xl, xr], axis=1)


def _nd_specs(bn):
    return ([pl.BlockSpec((bn, 16), lambda i: (i, 0))] * 4
            + [pl.BlockSpec((bn, 1), lambda i: (i, 0))] * 4)


def _fin_xform(nums, dens, bias, wlT, wrT, bl, br):
    bn = 1024
    grid = (_NP // bn,)
    return pl.pallas_call(
        _fin_xform_body,
        grid=grid,
        in_specs=_nd_specs(bn) + [
            pl.BlockSpec((1, _H), lambda i: (0, 0)),
            pl.BlockSpec((_H, _H), lambda i: (0, 0)),
            pl.BlockSpec((_H, _H), lambda i: (0, 0)),
            pl.BlockSpec((1, _H), lambda i: (0, 0)),
            pl.BlockSpec((1, _H), lambda i: (0, 0)),
        ],
        out_specs=pl.BlockSpec((bn, 2 * _H), lambda i: (i, 0)),
        out_shape=jax.ShapeDtypeStruct((_NP, 2 * _H), _f32),
    )(*nums, *dens, bias, wlT, wrT, bl, br)


def _fin_only_body(n0, n1, n2, n3, d0, d1, d2, d3, bias_ref, x_ref):
    x = _fin_x([n0[...], n1[...], n2[...], n3[...]],
               [d0[...], d1[...], d2[...], d3[...]], bias_ref[...])
    bn = x.shape[0]
    x_ref[...] = jnp.concatenate([x, jnp.zeros((bn, _H), _f32)], axis=1)


def _fin_only(nums, dens, bias):
    bn = 1024
    grid = (_NP // bn,)
    return pl.pallas_call(
        _fin_only_body,
        grid=grid,
        in_specs=_nd_specs(bn) + [pl.BlockSpec((1, _H), lambda i: (0, 0))],
        out_specs=pl.BlockSpec((bn, 2 * _H), lambda i: (i, 0)),
        out_shape=jax.ShapeDtypeStruct((_NP, 2 * _H), _f32),
    )(*nums, *dens, bias)


def _edge_sc(tab, edges, att_flat, znum, zden, hb):
    mesh = plsc.VectorSubcoreMesh(core_axis_name="c", subcore_axis_name="s")

    @functools.partial(
        pl.kernel,
        mesh=mesh,
        compiler_params=pltpu.CompilerParams(
            needs_layout_passes=False, use_tc_tiling_on_sc=False),
        out_type=[
            jax.ShapeDtypeStruct((_NP, 16), _f32),
            jax.ShapeDtypeStruct((_NP, 16), _f32),
            jax.ShapeDtypeStruct((_NP // 16, 16), _f32),
            jax.ShapeDtypeStruct((_NP // 16, 16), _f32),
        ],
        scratch_types=[
            pltpu.VMEM((16,), _i32),
            pltpu.VMEM((16, _K), _i32),
            pltpu.VMEM((_K,), _i32),
            pltpu.VMEM((_K, 2 * _H), _f32),
            pltpu.VMEM((_K, 2 * _H), _f32),
            pltpu.VMEM((_K, 2 * _H), _f32),
            pltpu.VMEM((_K, 2 * _H), _f32),
            pltpu.VMEM((_K, 16), _f32),
            pltpu.VMEM((_K, 16), _f32),
            pltpu.VMEM((_K, 16), _f32),
            pltpu.VMEM((_K, 16), _f32),
            pltpu.VMEM((_K,), _i32),
            pltpu.VMEM((64,), _f32),
            pltpu.VMEM_SHARED((_NP, 16), _f32),
            pltpu.VMEM_SHARED((_NP // 16, 16), _f32),
            pltpu.SemaphoreType.DMA,
            pltpu.SemaphoreType.DMA,
            pltpu.SemaphoreType.DMA,
            pltpu.SemaphoreType.DMA,
            pltpu.SemaphoreType.DMA,
            pltpu.SemaphoreType.DMA,
            pltpu.SemaphoreType.DMA,
            pltpu.SemaphoreType.DMA,
            pltpu.SemaphoreType.DMA,
        ],
    )
    def body(tab_h, edges_h, att_h, znum_h, zden_h,
             num_a_o, num_b_o, den_a_o, den_b_o,
             rowidx, ebuf, dstdv, xsbuf0, xdbuf0, xsbuf1, xdbuf1,
             msgbuf0, denbuf0, msgbuf1, denbuf1, dstdv1, attv,
             num_sh, den_sh, sem_a0, sem_b0, sem_a1, sem_b1,
             sem_m0, sem_m1, sem_d0, sem_d1, sem_e):
        c = lax.axis_index("c")
        s = lax.axis_index("s")
        row0 = s * _RPT
        drow0 = s * (_RPT // 16)

        pltpu.sync_copy(znum_h, num_sh.at[pl.ds(row0, _RPT)])
        pltpu.sync_copy(zden_h, den_sh.at[pl.ds(drow0, _RPT // 16)])
        pltpu.sync_copy(att_h, attv)
        plsc.subcore_barrier()

        iota = lax.iota(_i32, 16)
        colj = (hb + c) * 16
        att_a = plsc.load_gather(attv, [iota + colj])
        zero16 = jnp.zeros((16,), _f32)
        nrows = _E2P // _K

        def superchunk(g8, carry):
            base = s * _NCH + g8 * 8
            vals = (base + jnp.bitwise_and(iota, 7)
                    + jnp.where(iota >= 8, nrows, 0))
            plsc.store_scatter(rowidx, [iota], vals)
            pltpu.async_copy(edges_h.at[rowidx], ebuf, sem_e).wait()

            bufs = [(xsbuf0, xdbuf0, sem_a0, sem_b0),
                    (xsbuf1, xdbuf1, sem_a1, sem_b1)]
            obufs = [(msgbuf0, denbuf0, dstdv, sem_m0, sem_d0),
                     (msgbuf1, denbuf1, dstdv1, sem_m1, sem_d1)]

            def issue(j):
                xs, xd, sa, sb = bufs[j % 2]
                cj = pltpu.async_copy(tab_h.at[ebuf.at[j]], xs, sa)
                ci = pltpu.async_copy(tab_h.at[ebuf.at[8 + j]], xd, sb)
                return cj, ci

            pending = issue(0)
            sc_pend = [None, None]
            for j in range(8):
                xs, xd, _, _ = bufs[j % 2]
                msgb, denb, ddv, semm, semd = obufs[j % 2]
                rdst = jnp.full((16,), 8 + j, _i32)
                pending[0].wait()
                pending[1].wait()
                if j < 7:
                    pending = issue(j + 1)
                if sc_pend[j % 2] is not None:
                    sc_pend[j % 2][0].wait()
                    sc_pend[j % 2][1].wait()

                def dshift(j2, carry2, rdst=rdst, ddv=ddv):
                    ix = j2 * 16 + iota
                    v = plsc.load_gather(ebuf, [rdst, ix])
                    plsc.store_scatter(ddv, [ix],
                                       lax.shift_right_logical(v, 4))
                    return carry2

                lax.fori_loop(0, _K // 16, dshift, 0)

                def edge(k, carry2, xs=xs, xd=xd, rdst=rdst,
                         msgb=msgb, denb=denb):
                    rk = jnp.full((16,), k, _i32)
                    xj0 = plsc.load_gather(xs, [rk, iota + colj])
                    xi0 = plsc.load_gather(xd, [rk, iota + 64 + colj])
                    t0 = xi0 + xj0
                    l0 = jnp.sum(jnp.maximum(t0, 0.2 * t0) * att_a)
                    a0 = jnp.exp(jnp.full((16,), l0, _f32))
                    plsc.store_scatter(msgb, [rk, iota], xj0 * a0)
                    dk = plsc.load_gather(ebuf, [rdst, rk])
                    p0 = jnp.bitwise_and(dk, 15)
                    d = jnp.where(iota == p0, a0, zero16)
                    plsc.store_scatter(denb, [rk, iota], d)
                    return carry2

                lax.fori_loop(0, _K, edge, 0)
                c1 = pltpu.async_copy(msgb, num_sh.at[ebuf.at[8 + j]],
                                      semm, add=True)
                c2 = pltpu.async_copy(denb, den_sh.at[ddv],
                                      semd, add=True)
                sc_pend[j % 2] = (c1, c2)
            for p in sc_pend:
                if p is not None:
                    p[0].wait()
                    p[1].wait()
            return carry

        lax.fori_loop(0, _NCH // 8, superchunk, 0)
        plsc.subcore_barrier()

        @pl.when(c == 0)
        def _():
            pltpu.sync_copy(num_sh.at[pl.ds(row0, _RPT)],
                            num_a_o.at[pl.ds(row0, _RPT)])
            pltpu.sync_copy(den_sh.at[pl.ds(drow0, _RPT // 16)],
                            den_a_o.at[pl.ds(drow0, _RPT // 16)])

        @pl.when(c == 1)
        def _():
            pltpu.sync_copy(num_sh.at[pl.ds(row0, _RPT)],
                            num_b_o.at[pl.ds(row0, _RPT)])
            pltpu.sync_copy(den_sh.at[pl.ds(drow0, _RPT // 16)],
                            den_b_o.at[pl.ds(drow0, _RPT // 16)])

    return body(tab, edges, att_flat, znum, zden)


def _gat_layer(tab, edges, att_flat, znum, zden):
    n0, n1, d0, d1 = _edge_sc(tab, edges, att_flat, znum, zden, 0)
    att2, _ = lax.optimization_barrier((att_flat, n0))
    n2, n3, d2, d3 = _edge_sc(tab, edges, att2, znum, zden, 2)
    nums = [n0, n1, n2, n3]
    dens = [d.reshape(_NP, 1) for d in (d0, d1, d2, d3)]
    return nums, dens


def _pool_sc(x2, batch_pad, focal, zsc):
    mesh = plsc.VectorSubcoreMesh(core_axis_name="c", subcore_axis_name="s")

    @functools.partial(
        pl.kernel,
        mesh=mesh,
        compiler_params=pltpu.CompilerParams(
            needs_layout_passes=False, use_tc_tiling_on_sc=False),
        out_type=[
            jax.ShapeDtypeStruct((_GP, 2 * _H), _f32),
            jax.ShapeDtypeStruct((_GP, 2 * _H), _f32),
            jax.ShapeDtypeStruct((_G, 2 * _H), _f32),
        ],
        scratch_types=[
            pltpu.VMEM((_PC,), _i32),
            pltpu.VMEM((_PC,), _i32),
            pltpu.VMEM((_PC, 2 * _H), _f32),
            pltpu.VMEM((32,), _i32),
            pltpu.VMEM((32, 2 * _H), _f32),
            pltpu.VMEM_SHARED((_GP, 2 * _H), _f32),
            pltpu.SemaphoreType.DMA,
        ],
    )
    def body(x2_h, batch_h, focal_h, zsc_h,
             scene0_o, scene1_o, agent_o,
             segv, rbuf, rowbuf, fidxv, agbuf, scene_sh, sem):
        c = lax.axis_index("c")
        s = lax.axis_index("s")
        wid = s * 2 + c
        rows = _GP // 16
        zrow0 = s * rows
        pltpu.sync_copy(zsc_h, scene_sh.at[pl.ds(zrow0, rows)])
        plsc.subcore_barrier()
        iota = lax.iota(_i32, 16)

        def chunk(g, carry):
            base = (wid * _NPC + g) * _PC
            pltpu.sync_copy(batch_h.at[pl.ds(base, _PC)], segv)

            def fill(j, carry2):
                ix = j * 16 + iota
                plsc.store_scatter(rbuf, [ix], base + ix)
                return carry2

            lax.fori_loop(0, _PC // 16, fill, 0)
            pltpu.async_copy(x2_h.at[rbuf], rowbuf, sem).wait()
            pltpu.sync_copy(rowbuf, scene_sh.at[segv], add=True)
            return carry

        lax.fori_loop(0, _NPC, chunk, 0)

        pltpu.sync_copy(focal_h.at[pl.ds(wid * 32, 32)], fidxv)
        pltpu.async_copy(x2_h.at[fidxv], agbuf, sem).wait()
        pltpu.sync_copy(agbuf, agent_o.at[pl.ds(wid * 32, 32)])

        plsc.subcore_barrier()

        @pl.when(c == 0)
        def _():
            pltpu.sync_copy(scene_sh.at[pl.ds(zrow0, rows)],
                            scene0_o.at[pl.ds(zrow0, rows)])

        @pl.when(c == 1)
        def _():
            pltpu.sync_copy(scene_sh.at[pl.ds(zrow0, rows)],
                            scene1_o.at[pl.ds(zrow0, rows)])

    return body(x2, batch_pad, focal, zsc)


def _dec_body(ag_ref, s0_ref, s1_ref, w1_ref, b1_ref, w2_ref, b2_ref, o_ref):
    ag = ag_ref[...]
    sc = s0_ref[...] + s1_ref[...]
    fused = jnp.concatenate([ag, sc], axis=1)
    hid = jax.nn.relu(jnp.dot(fused, w1_ref[...],
                              preferred_element_type=_f32) + b1_ref[...])
    o_ref[...] = jnp.dot(hid, w2_ref[...],
                         preferred_element_type=_f32) + b2_ref[...]


def _decoder(agent, scene0, scene1, w1T, b1, w2T, b2):
    return pl.pallas_call(
        _dec_body,
        grid=(1,),
        in_specs=[
            pl.BlockSpec((_G, _H), lambda i: (0, 0)),
            pl.BlockSpec((_G, _H), lambda i: (0, 0)),
            pl.BlockSpec((_G, _H), lambda i: (0, 0)),
            pl.BlockSpec((2 * _H, 2 * _H), lambda i: (0, 0)),
            pl.BlockSpec((1, 2 * _H), lambda i: (0, 0)),
            pl.BlockSpec((2 * _H, _HOR * 2), lambda i: (0, 0)),
            pl.BlockSpec((1, _HOR * 2), lambda i: (0, 0)),
        ],
        out_specs=pl.BlockSpec((_G, _HOR * 2), lambda i: (0, 0)),
        out_shape=jax.ShapeDtypeStruct((_G, _HOR * 2), _f32),
    )(agent, scene0, scene1, w1T, b1, w2T, b2)


def kernel(history, lstm_W_ih, lstm_W_hh, lstm_b_ih, lstm_b_hh,
           g1_Wl, g1_bl, g1_Wr, g1_br, g1_att, g1_bias,
           g2_Wl, g2_bl, g2_Wr, g2_br, g2_att, g2_bias,
           dec_W1, dec_b1, dec_W2, dec_b2,
           edge_index, batch, focal_agent_index):
    f32 = _f32

    hist = history.reshape(_N, 2 * _T).astype(f32)
    hist = jnp.pad(hist, ((0, _NP - _N), (0, 0)))
    b_lstm = (lstm_b_ih + lstm_b_hh).reshape(1, 4 * _H).astype(f32)
    h = _lstm(hist, lstm_W_ih.T.astype(f32), lstm_W_hh.T.astype(f32), b_lstm)

    npad = _E2P - _N - edge_index.shape[1]
    src = jnp.concatenate([
        edge_index[0].astype(_i32),
        jnp.arange(_N, dtype=_i32),
        jnp.full((npad,), _JUNK, _i32),
    ])
    dst = jnp.concatenate([
        edge_index[1].astype(_i32),
        jnp.arange(_N, dtype=_i32),
        jnp.full((npad,), _JUNK, _i32),
    ])
    edges = jnp.concatenate([src, dst]).reshape(2 * _E2P // _K, _K)

    znum = jnp.zeros((_RPT, 16), f32)
    zden = jnp.zeros((_RPT // 16, 16), f32)

    tab1 = _xform(
        h, g1_Wl.T.astype(f32), g1_Wr.T.astype(f32),
        g1_bl.reshape(1, _H).astype(f32), g1_br.reshape(1, _H).astype(f32))
    nums1, dens1 = _gat_layer(tab1, edges, g1_att.reshape(-1).astype(f32),
                              znum, zden)

    tab2 = _fin_xform(
        nums1, dens1, g1_bias.reshape(1, _H).astype(f32),
        g2_Wl.T.astype(f32), g2_Wr.T.astype(f32),
        g2_bl.reshape(1, _H).astype(f32), g2_br.reshape(1, _H).astype(f32))

    nums2, dens2 = _gat_layer(tab2, edges, g2_att.reshape(-1).astype(f32),
                              znum, zden)
    x2 = _fin_only(nums2, dens2, g2_bias.reshape(1, _H).astype(f32))

    batch_pad = jnp.concatenate([
        batch.astype(_i32), jnp.full((_NP - _N,), _G, _i32)])
    zsc = jnp.zeros((_GP // 16, 2 * _H), f32)
    scene0, scene1, agent = _pool_sc(
        x2, batch_pad, focal_agent_index.astype(_i32), zsc)

    coords = _decoder(
        agent[:, :_H], scene0[:_G, :_H], scene1[:_G, :_H],
        dec_W1.T.astype(f32), dec_b1.reshape(1, -1).astype(f32),
        dec_W2.T.astype(f32), dec_b2.reshape(1, -1).astype(f32))
    return coords.reshape(_G, _HOR, 2)


# speedup vs baseline: 36.8971x; 1.5465x over previous
import functools

import jax
import jax.numpy as jnp
from jax import lax
from jax.experimental import pallas as pl
from jax.experimental.pallas import tpu as pltpu
from jax.experimental.pallas import tpu_sc as plsc

_N = 50000
_T = 20
_H = 64
_G = 1024
_HOR = 30

_NP = 51200
_JUNK = _N
_K = 128
_NCH = 416
_E2P = 16 * _K * _NCH
_RPT = _NP // 16

_GP = 1152
_PC = 64
_NPC = _NP // (32 * _PC)

_f32 = jnp.float32
_i32 = jnp.int32


def _lstm_body(hist_ref, wih_ref, whh_ref, b_ref, h_ref):
    hist = hist_ref[...]
    wih = wih_ref[...]
    whh = whh_ref[...]
    b = b_ref[...]
    bn = hist.shape[0]
    h = jnp.zeros((bn, _H), _f32)
    c = jnp.zeros((bn, _H), _f32)
    for t in range(_T):
        x0 = hist[:, 2 * t:2 * t + 1]
        x1 = hist[:, 2 * t + 1:2 * t + 2]
        gates = (x0 * wih[0:1, :] + x1 * wih[1:2, :]
                 + jnp.dot(h, whh, preferred_element_type=_f32) + b)
        i = jax.nn.sigmoid(gates[:, 0:_H])
        f = jax.nn.sigmoid(gates[:, _H:2 * _H])
        g = jnp.tanh(gates[:, 2 * _H:3 * _H])
        o = jax.nn.sigmoid(gates[:, 3 * _H:4 * _H])
        c = f * c + i * g
        h = o * jnp.tanh(c)
    h_ref[...] = h


def _lstm(hist_pad, wihT, whhT, b):
    bn = 1024
    grid = (_NP // bn,)
    return pl.pallas_call(
        _lstm_body,
        grid=grid,
        in_specs=[
            pl.BlockSpec((bn, 2 * _T), lambda i: (i, 0)),
            pl.BlockSpec((2, 4 * _H), lambda i: (0, 0)),
            pl.BlockSpec((_H, 4 * _H), lambda i: (0, 0)),
            pl.BlockSpec((1, 4 * _H), lambda i: (0, 0)),
        ],
        out_specs=pl.BlockSpec((bn, _H), lambda i: (i, 0)),
        out_shape=jax.ShapeDtypeStruct((_NP, _H), _f32),
    )(hist_pad, wihT, whhT, b)


def _xform_body(x_ref, wl_ref, wr_ref, bl_ref, br_ref, tab_ref):
    x = x_ref[...]
    xl = jnp.dot(x, wl_ref[...], preferred_element_type=_f32) + bl_ref[...]
    xr = jnp.dot(x, wr_ref[...], preferred_element_type=_f32) + br_ref[...]
    tab_ref[...] = jnp.concatenate([xl, xr], axis=1)


def _xform(x, wlT, wrT, bl, br):
    bn = 1024
    grid = (_NP // bn,)
    return pl.pallas_call(
        _xform_body,
        grid=grid,
        in_specs=[
            pl.BlockSpec((bn, _H), lambda i: (i, 0)),
            pl.BlockSpec((_H, _H), lambda i: (0, 0)),
            pl.BlockSpec((_H, _H), lambda i: (0, 0)),
            pl.BlockSpec((1, _H), lambda i: (0, 0)),
            pl.BlockSpec((1, _H), lambda i: (0, 0)),
        ],
        out_specs=pl.BlockSpec((bn, 2 * _H), lambda i: (i, 0)),
        out_shape=jax.ShapeDtypeStruct((_NP, 2 * _H), _f32),
    )(x, wlT, wrT, bl, br)


def _fin_x(nums, dens, bias):
    parts = [n / (d + 1e-16) for n, d in zip(nums, dens)]
    return jax.nn.relu(jnp.concatenate(parts, axis=1) + bias)


def _fin_xform_body(n0, n1, n2, n3, d0, d1, d2, d3, bias_ref,
                    wl_ref, wr_ref, bl_ref, br_ref, tab_ref):
    x = _fin_x([n0[...], n1[...], n2[...], n3[...]],
               [d0[...], d1[...], d2[...], d3[...]], bias_ref[...])
    xl = jnp.dot(x, wl_ref[...], preferred_element_type=_f32) + bl_ref[...]
    xr = jnp.dot(x, wr_ref[...], preferred_element_type=_f32) + br_ref[...]
    tab_ref[...] = jnp.concatenate([xl, xr], axis=1)


def _nd_specs(bn):
    return ([pl.BlockSpec((bn, 16), lambda i: (i, 0))] * 4
            + [pl.BlockSpec((bn, 1), lambda i: (i, 0))] * 4)


def _fin_xform(nums, dens, bias, wlT, wrT, bl, br):
    bn = 1024
    grid = (_NP // bn,)
    return pl.pallas_call(
        _fin_xform_body,
        grid=grid,
        in_specs=_nd_specs(bn) + [
            pl.BlockSpec((1, _H), lambda i: (0, 0)),
            pl.BlockSpec((_H, _H), lambda i: (0, 0)),
            pl.BlockSpec((_H, _H), lambda i: (0, 0)),
            pl.BlockSpec((1, _H), lambda i: (0, 0)),
            pl.BlockSpec((1, _H), lambda i: (0, 0)),
        ],
        out_specs=pl.BlockSpec((bn, 2 * _H), lambda i: (i, 0)),
        out_shape=jax.ShapeDtypeStruct((_NP, 2 * _H), _f32),
    )(*nums, *dens, bias, wlT, wrT, bl, br)


def _fin_only_body(n0, n1, n2, n3, d0, d1, d2, d3, bias_ref, x_ref):
    x = _fin_x([n0[...], n1[...], n2[...], n3[...]],
               [d0[...], d1[...], d2[...], d3[...]], bias_ref[...])
    bn = x.shape[0]
    x_ref[...] = jnp.concatenate([x, jnp.zeros((bn, _H), _f32)], axis=1)


def _fin_only(nums, dens, bias):
    bn = 1024
    grid = (_NP // bn,)
    return pl.pallas_call(
        _fin_only_body,
        grid=grid,
        in_specs=_nd_specs(bn) + [pl.BlockSpec((1, _H), lambda i: (0, 0))],
        out_specs=pl.BlockSpec((bn, 2 * _H), lambda i: (i, 0)),
        out_shape=jax.ShapeDtypeStruct((_NP, 2 * _H), _f32),
    )(*nums, *dens, bias)


def _edge_sc(tab, edges, att_flat, znum, zden, hb):
    mesh = plsc.VectorSubcoreMesh(core_axis_name="c", subcore_axis_name="s")

    @functools.partial(
        pl.kernel,
        mesh=mesh,
        compiler_params=pltpu.CompilerParams(
            needs_layout_passes=False, use_tc_tiling_on_sc=False),
        out_type=[
            jax.ShapeDtypeStruct((_NP, 16), _f32),
            jax.ShapeDtypeStruct((_NP, 16), _f32),
            jax.ShapeDtypeStruct((_NP // 16, 16), _f32),
            jax.ShapeDtypeStruct((_NP // 16, 16), _f32),
        ],
        scratch_types=[
            pltpu.VMEM((16,), _i32),
            pltpu.VMEM((16, _K), _i32),
            pltpu.VMEM((_K,), _i32),
            pltpu.VMEM((_K, 2 * _H), _f32),
            pltpu.VMEM((_K, 2 * _H), _f32),
            pltpu.VMEM((_K, 2 * _H), _f32),
            pltpu.VMEM((_K, 2 * _H), _f32),
            pltpu.VMEM((_K, 16), _f32),
            pltpu.VMEM((_K, 16), _f32),
            pltpu.VMEM((_K, 16), _f32),
            pltpu.VMEM((_K, 16), _f32),
            pltpu.VMEM((_K,), _i32),
            pltpu.VMEM((64,), _f32),
            pltpu.VMEM_SHARED((_NP, 16), _f32),
            pltpu.VMEM_SHARED((_NP // 16, 16), _f32),
            pltpu.SemaphoreType.DMA,
            pltpu.SemaphoreType.DMA,
            pltpu.SemaphoreType.DMA,
            pltpu.SemaphoreType.DMA,
            pltpu.SemaphoreType.DMA,
            pltpu.SemaphoreType.DMA,
            pltpu.SemaphoreType.DMA,
            pltpu.SemaphoreType.DMA,
            pltpu.SemaphoreType.DMA,
        ],
    )
    def body(tab_h, edges_h, att_h, znum_h, zden_h,
             num_a_o, num_b_o, den_a_o, den_b_o,
             rowidx, ebuf, dstdv, xsbuf0, xdbuf0, xsbuf1, xdbuf1,
             msgbuf0, denbuf0, msgbuf1, denbuf1, dstdv1, attv,
             num_sh, den_sh, sem_a0, sem_b0, sem_a1, sem_b1,
             sem_m0, sem_m1, sem_d0, sem_d1, sem_e):
        c = lax.axis_index("c")
        s = lax.axis_index("s")
        row0 = s * _RPT
        drow0 = s * (_RPT // 16)

        pltpu.sync_copy(znum_h, num_sh.at[pl.ds(row0, _RPT)])
        pltpu.sync_copy(zden_h, den_sh.at[pl.ds(drow0, _RPT // 16)])
        pltpu.sync_copy(att_h, attv)
        plsc.subcore_barrier()

        iota = lax.iota(_i32, 16)
        colj = (hb + c) * 16
        att_a = plsc.load_gather(attv, [iota + colj])
        zero16 = jnp.zeros((16,), _f32)
        nrows = _E2P // _K

        def superchunk(g8, carry):
            base = s * _NCH + g8 * 8
            vals = (base + jnp.bitwise_and(iota, 7)
                    + jnp.where(iota >= 8, nrows, 0))
            plsc.store_scatter(rowidx, [iota], vals)
            pltpu.async_copy(edges_h.at[rowidx], ebuf, sem_e).wait()

            bufs = [(xsbuf0, xdbuf0, sem_a0, sem_b0),
                    (xsbuf1, xdbuf1, sem_a1, sem_b1)]
            obufs = [(msgbuf0, denbuf0, dstdv, sem_m0, sem_d0),
                     (msgbuf1, denbuf1, dstdv1, sem_m1, sem_d1)]

            def issue(j):
                xs, xd, sa, sb = bufs[j % 2]
                cj = pltpu.async_copy(tab_h.at[ebuf.at[j]], xs, sa)
                ci = pltpu.async_copy(tab_h.at[ebuf.at[8 + j]], xd, sb)
                return cj, ci

            pending = issue(0)
            sc_pend = [None, None]
            for j in range(8):
                xs, xd, _, _ = bufs[j % 2]
                msgb, denb, ddv, semm, semd = obufs[j % 2]
                rdst = jnp.full((16,), 8 + j, _i32)
                pending[0].wait()
                pending[1].wait()
                if j < 7:
                    pending = issue(j + 1)
                if sc_pend[j % 2] is not None:
                    sc_pend[j % 2][0].wait()
                    sc_pend[j % 2][1].wait()

                def dshift(j2, carry2, rdst=rdst, ddv=ddv):
                    ix = j2 * 16 + iota
                    v = plsc.load_gather(ebuf, [rdst, ix])
                    plsc.store_scatter(ddv, [ix],
                                       lax.shift_right_logical(v, 4))
                    return carry2

                lax.fori_loop(0, _K // 16, dshift, 0)

                @functools.partial(plsc.parallel_loop, 0, _K, unroll=4)
                def _(k, xs=xs, xd=xd, rdst=rdst, msgb=msgb, denb=denb):
                    rk = jnp.full((16,), k, _i32)
                    xj0 = plsc.load_gather(xs, [rk, iota + colj])
                    xi0 = plsc.load_gather(xd, [rk, iota + 64 + colj])
                    t0 = xi0 + xj0
                    l0 = jnp.sum(jnp.maximum(t0, 0.2 * t0) * att_a)
                    a0 = jnp.exp(jnp.full((16,), l0, _f32))
                    plsc.store_scatter(msgb, [rk, iota], xj0 * a0)
                    dk = plsc.load_gather(ebuf, [rdst, rk])
                    p0 = jnp.bitwise_and(dk, 15)
                    d = jnp.where(iota == p0, a0, zero16)
                    plsc.store_scatter(denb, [rk, iota], d)
                c1 = pltpu.async_copy(msgb, num_sh.at[ebuf.at[8 + j]],
                                      semm, add=True)
                c2 = pltpu.async_copy(denb, den_sh.at[ddv],
                                      semd, add=True)
                sc_pend[j % 2] = (c1, c2)
            for p in sc_pend:
                if p is not None:
                    p[0].wait()
                    p[1].wait()
            return carry

        lax.fori_loop(0, _NCH // 8, superchunk, 0)
        plsc.subcore_barrier()

        @pl.when(c == 0)
        def _():
            pltpu.sync_copy(num_sh.at[pl.ds(row0, _RPT)],
                            num_a_o.at[pl.ds(row0, _RPT)])
            pltpu.sync_copy(den_sh.at[pl.ds(drow0, _RPT // 16)],
                            den_a_o.at[pl.ds(drow0, _RPT // 16)])

        @pl.when(c == 1)
        def _():
            pltpu.sync_copy(num_sh.at[pl.ds(row0, _RPT)],
                            num_b_o.at[pl.ds(row0, _RPT)])
            pltpu.sync_copy(den_sh.at[pl.ds(drow0, _RPT // 16)],
                            den_b_o.at[pl.ds(drow0, _RPT // 16)])

    return body(tab, edges, att_flat, znum, zden)


def _gat_layer(tab, edges, att_flat, znum, zden):
    n0, n1, d0, d1 = _edge_sc(tab, edges, att_flat, znum, zden, 0)
    att2, _ = lax.optimization_barrier((att_flat, n0))
    n2, n3, d2, d3 = _edge_sc(tab, edges, att2, znum, zden, 2)
    nums = [n0, n1, n2, n3]
    dens = [d.reshape(_NP, 1) for d in (d0, d1, d2, d3)]
    return nums, dens


def _pool_sc(x2, batch_pad, focal, zsc):
    mesh = plsc.VectorSubcoreMesh(core_axis_name="c", subcore_axis_name="s")

    @functools.partial(
        pl.kernel,
        mesh=mesh,
        compiler_params=pltpu.CompilerParams(
            needs_layout_passes=False, use_tc_tiling_on_sc=False),
        out_type=[
            jax.ShapeDtypeStruct((_GP, 2 * _H), _f32),
            jax.ShapeDtypeStruct((_GP, 2 * _H), _f32),
            jax.ShapeDtypeStruct((_G, 2 * _H), _f32),
        ],
        scratch_types=[
            pltpu.VMEM((_PC,), _i32),
            pltpu.VMEM((_PC,), _i32),
            pltpu.VMEM((_PC, 2 * _H), _f32),
            pltpu.VMEM((32,), _i32),
            pltpu.VMEM((32, 2 * _H), _f32),
            pltpu.VMEM_SHARED((_GP, 2 * _H), _f32),
            pltpu.SemaphoreType.DMA,
        ],
    )
    def body(x2_h, batch_h, focal_h, zsc_h,
             scene0_o, scene1_o, agent_o,
             segv, rbuf, rowbuf, fidxv, agbuf, scene_sh, sem):
        c = lax.axis_index("c")
        s = lax.axis_index("s")
        wid = s * 2 + c
        rows = _GP // 16
        zrow0 = s * rows
        pltpu.sync_copy(zsc_h, scene_sh.at[pl.ds(zrow0, rows)])
        plsc.subcore_barrier()
        iota = lax.iota(_i32, 16)

        def chunk(g, carry):
            base = (wid * _NPC + g) * _PC
            pltpu.sync_copy(batch_h.at[pl.ds(base, _PC)], segv)

            def fill(j, carry2):
                ix = j * 16 + iota
                plsc.store_scatter(rbuf, [ix], base + ix)
                return carry2

            lax.fori_loop(0, _PC // 16, fill, 0)
            pltpu.async_copy(x2_h.at[rbuf], rowbuf, sem).wait()
            pltpu.sync_copy(rowbuf, scene_sh.at[segv], add=True)
            return carry

        lax.fori_loop(0, _NPC, chunk, 0)

        pltpu.sync_copy(focal_h.at[pl.ds(wid * 32, 32)], fidxv)
        pltpu.async_copy(x2_h.at[fidxv], agbuf, sem).wait()
        pltpu.sync_copy(agbuf, agent_o.at[pl.ds(wid * 32, 32)])

        plsc.subcore_barrier()

        @pl.when(c == 0)
        def _():
            pltpu.sync_copy(scene_sh.at[pl.ds(zrow0, rows)],
                            scene0_o.at[pl.ds(zrow0, rows)])

        @pl.when(c == 1)
        def _():
            pltpu.sync_copy(scene_sh.at[pl.ds(zrow0, rows)],
                            scene1_o.at[pl.ds(zrow0, rows)])

    return body(x2, batch_pad, focal, zsc)


def _dec_body(ag_ref, s0_ref, s1_ref, w1_ref, b1_ref, w2_ref, b2_ref, o_ref):
    ag = ag_ref[...]
    sc = s0_ref[...] + s1_ref[...]
    fused = jnp.concatenate([ag, sc], axis=1)
    hid = jax.nn.relu(jnp.dot(fused, w1_ref[...],
                              preferred_element_type=_f32) + b1_ref[...])
    o_ref[...] = jnp.dot(hid, w2_ref[...],
                         preferred_element_type=_f32) + b2_ref[...]


def _decoder(agent, scene0, scene1, w1T, b1, w2T, b2):
    return pl.pallas_call(
        _dec_body,
        grid=(1,),
        in_specs=[
            pl.BlockSpec((_G, _H), lambda i: (0, 0)),
            pl.BlockSpec((_G, _H), lambda i: (0, 0)),
            pl.BlockSpec((_G, _H), lambda i: (0, 0)),
            pl.BlockSpec((2 * _H, 2 * _H), lambda i: (0, 0)),
            pl.BlockSpec((1, 2 * _H), lambda i: (0, 0)),
            pl.BlockSpec((2 * _H, _HOR * 2), lambda i: (0, 0)),
            pl.BlockSpec((1, _HOR * 2), lambda i: (0, 0)),
        ],
        out_specs=pl.BlockSpec((_G, _HOR * 2), lambda i: (0, 0)),
        out_shape=jax.ShapeDtypeStruct((_G, _HOR * 2), _f32),
    )(agent, scene0, scene1, w1T, b1, w2T, b2)


def kernel(history, lstm_W_ih, lstm_W_hh, lstm_b_ih, lstm_b_hh,
           g1_Wl, g1_bl, g1_Wr, g1_br, g1_att, g1_bias,
           g2_Wl, g2_bl, g2_Wr, g2_br, g2_att, g2_bias,
           dec_W1, dec_b1, dec_W2, dec_b2,
           edge_index, batch, focal_agent_index):
    f32 = _f32

    hist = history.reshape(_N, 2 * _T).astype(f32)
    hist = jnp.pad(hist, ((0, _NP - _N), (0, 0)))
    b_lstm = (lstm_b_ih + lstm_b_hh).reshape(1, 4 * _H).astype(f32)
    h = _lstm(hist, lstm_W_ih.T.astype(f32), lstm_W_hh.T.astype(f32), b_lstm)

    npad = _E2P - _N - edge_index.shape[1]
    src = jnp.concatenate([
        edge_index[0].astype(_i32),
        jnp.arange(_N, dtype=_i32),
        jnp.full((npad,), _JUNK, _i32),
    ])
    dst = jnp.concatenate([
        edge_index[1].astype(_i32),
        jnp.arange(_N, dtype=_i32),
        jnp.full((npad,), _JUNK, _i32),
    ])
    edges = jnp.concatenate([src, dst]).reshape(2 * _E2P // _K, _K)

    znum = jnp.zeros((_RPT, 16), f32)
    zden = jnp.zeros((_RPT // 16, 16), f32)

    tab1 = _xform(
        h, g1_Wl.T.astype(f32), g1_Wr.T.astype(f32),
        g1_bl.reshape(1, _H).astype(f32), g1_br.reshape(1, _H).astype(f32))
    nums1, dens1 = _gat_layer(tab1, edges, g1_att.reshape(-1).astype(f32),
                              znum, zden)

    tab2 = _fin_xform(
        nums1, dens1, g1_bias.reshape(1, _H).astype(f32),
        g2_Wl.T.astype(f32), g2_Wr.T.astype(f32),
        g2_bl.reshape(1, _H).astype(f32), g2_br.reshape(1, _H).astype(f32))

    nums2, dens2 = _gat_layer(tab2, edges, g2_att.reshape(-1).astype(f32),
                              znum, zden)
    x2 = _fin_only(nums2, dens2, g2_bias.reshape(1, _H).astype(f32))

    batch_pad = jnp.concatenate([
        batch.astype(_i32), jnp.full((_NP - _N,), _G, _i32)])
    zsc = jnp.zeros((_GP // 16, 2 * _H), f32)
    scene0, scene1, agent = _pool_sc(
        x2, batch_pad, focal_agent_index.astype(_i32), zsc)

    coords = _decoder(
        agent[:, :_H], scene0[:_G, :_H], scene1[:_G, :_H],
        dec_W1.T.astype(f32), dec_b1.reshape(1, -1).astype(f32),
        dec_W2.T.astype(f32), dec_b2.reshape(1, -1).astype(f32))
    return coords.reshape(_G, _HOR, 2)
